# baseline (speedup 1.0000x reference)
import functools

import jax
import jax.numpy as jnp
from jax import lax
from jax.experimental import pallas as pl
from jax.experimental.pallas import tpu as pltpu
from jax.experimental.pallas import tpu_sc as plsc

NC = 2
NS = 16
L = 16
NW = NC * NS

D = 128
C = 256


def _dist_mult_body(si_hbm, pi_hbm, oi_hbm, nodes_hbm, rel_hbm, out_hbm,
                    si_v, pi_v, oi_v, s_rows, p_rows, o_rows, out_v, sem):
    bpw = out_v.shape[0]
    nchunk = bpw // C
    wid = lax.axis_index("s") * NC + lax.axis_index("c")
    base = wid * bpw
    row_ids = lax.iota(jnp.int32, L)

    pltpu.sync_copy(si_hbm.at[pl.ds(base, bpw)], si_v)
    pltpu.sync_copy(pi_hbm.at[pl.ds(base, bpw)], pi_v)
    pltpu.sync_copy(oi_hbm.at[pl.ds(base, bpw)], oi_v)

    for c in range(nchunk):
        cp1 = pltpu.async_copy(nodes_hbm.at[si_v.at[pl.ds(c * C, C)]],
                               s_rows, sem)
        cp2 = pltpu.async_copy(rel_hbm.at[pi_v.at[pl.ds(c * C, C)]],
                               p_rows, sem)
        cp3 = pltpu.async_copy(nodes_hbm.at[oi_v.at[pl.ds(c * C, C)]],
                               o_rows, sem)
        cp1.wait()
        cp2.wait()
        cp3.wait()

        def group_body(g, carry, c=c):
            rows = row_ids + g * L
            accs = [jnp.zeros((L,), jnp.float32) for _ in range(4)]
            for d in range(D):
                cols = jnp.full((L,), d, jnp.int32)
                sv = plsc.load_gather(s_rows, [rows, cols])
                pv = plsc.load_gather(p_rows, [rows, cols])
                ov = plsc.load_gather(o_rows, [rows, cols])
                accs[d % 4] = accs[d % 4] + sv * pv * ov
            acc = (accs[0] + accs[1]) + (accs[2] + accs[3])
            out_v[pl.ds(c * C + g * L, L)] = acc
            return carry

        lax.fori_loop(0, C // L, group_body, 0)

    pltpu.sync_copy(out_v, out_hbm.at[pl.ds(base, bpw)])


def kernel(triples, nodes, relations):
    b = triples.shape[0]
    bpw = b // NW
    si = triples[:, 0].astype(jnp.int32)
    pi = triples[:, 1].astype(jnp.int32)
    oi = triples[:, 2].astype(jnp.int32)

    mesh = plsc.VectorSubcoreMesh(core_axis_name="c", subcore_axis_name="s")
    run = pl.kernel(
        _dist_mult_body,
        out_type=jax.ShapeDtypeStruct((b,), jnp.float32),
        mesh=mesh,
        compiler_params=pltpu.CompilerParams(needs_layout_passes=False),
        scratch_types=[
            pltpu.VMEM((b // NW,), jnp.int32),
            pltpu.VMEM((b // NW,), jnp.int32),
            pltpu.VMEM((b // NW,), jnp.int32),
            pltpu.VMEM((C, D), jnp.float32),
            pltpu.VMEM((C, D), jnp.float32),
            pltpu.VMEM((C, D), jnp.float32),
            pltpu.VMEM((b // NW,), jnp.float32),
            pltpu.SemaphoreType.DMA,
        ],
    )
    return run(si, pi, oi, nodes, relations)


# speedup vs baseline: 2.0516x; 2.0516x over previous
import functools

import jax
import jax.numpy as jnp
from jax import lax
from jax.experimental import pallas as pl
from jax.experimental.pallas import tpu as pltpu
from jax.experimental.pallas import tpu_sc as plsc

NC = 2
NS = 16
L = 16
NW = NC * NS

D = 128
C = 256


def _dist_mult_body(si_hbm, pi_hbm, oi_hbm, nodes_hbm, rel_hbm, out_hbm,
                    si_v, pi_v, oi_v, s_rows, p_rows, o_rows, out_v, sem):
    bpw = out_v.shape[0]
    nchunk = bpw // C
    wid = lax.axis_index("s") * NC + lax.axis_index("c")
    base = wid * bpw
    row_ids = lax.iota(jnp.int32, L)

    pltpu.sync_copy(si_hbm.at[pl.ds(base, bpw)], si_v)
    pltpu.sync_copy(pi_hbm.at[pl.ds(base, bpw)], pi_v)
    pltpu.sync_copy(oi_hbm.at[pl.ds(base, bpw)], oi_v)

    for c in range(nchunk):
        cp1 = pltpu.async_copy(nodes_hbm.at[si_v.at[pl.ds(c * C, C)]],
                               s_rows, sem)
        cp2 = pltpu.async_copy(rel_hbm.at[pi_v.at[pl.ds(c * C, C)]],
                               p_rows, sem)
        cp3 = pltpu.async_copy(nodes_hbm.at[oi_v.at[pl.ds(c * C, C)]],
                               o_rows, sem)
        cp1.wait()
        cp2.wait()
        cp3.wait()

        def group_body(g, carry, c=c):
            rows = row_ids + g * L
            accs = [jnp.zeros((L,), jnp.float32) for _ in range(4)]
            for d in range(D):
                cols = (row_ids + d) & (D - 1)
                sv = plsc.load_gather(s_rows, [rows, cols])
                pv = plsc.load_gather(p_rows, [rows, cols])
                ov = plsc.load_gather(o_rows, [rows, cols])
                accs[d % 4] = accs[d % 4] + sv * pv * ov
            acc = (accs[0] + accs[1]) + (accs[2] + accs[3])
            out_v[pl.ds(c * C + g * L, L)] = acc
            return carry

        lax.fori_loop(0, C // L, group_body, 0)

    pltpu.sync_copy(out_v, out_hbm.at[pl.ds(base, bpw)])


def kernel(triples, nodes, relations):
    b = triples.shape[0]
    bpw = b // NW
    si = triples[:, 0].astype(jnp.int32)
    pi = triples[:, 1].astype(jnp.int32)
    oi = triples[:, 2].astype(jnp.int32)

    mesh = plsc.VectorSubcoreMesh(core_axis_name="c", subcore_axis_name="s")
    run = pl.kernel(
        _dist_mult_body,
        out_type=jax.ShapeDtypeStruct((b,), jnp.float32),
        mesh=mesh,
        compiler_params=pltpu.CompilerParams(needs_layout_passes=False),
        scratch_types=[
            pltpu.VMEM((b // NW,), jnp.int32),
            pltpu.VMEM((b // NW,), jnp.int32),
            pltpu.VMEM((b // NW,), jnp.int32),
            pltpu.VMEM((C, D), jnp.float32),
            pltpu.VMEM((C, D), jnp.float32),
            pltpu.VMEM((C, D), jnp.float32),
            pltpu.VMEM((b // NW,), jnp.float32),
            pltpu.SemaphoreType.DMA,
        ],
    )
    return run(si, pi, oi, nodes, relations)


# speedup vs baseline: 4.0318x; 1.9653x over previous
import functools

import jax
import jax.numpy as jnp
from jax import lax
from jax.experimental import pallas as pl
from jax.experimental.pallas import tpu as pltpu
from jax.experimental.pallas import tpu_sc as plsc

NC = 2
NS = 16
L = 16
NW = NC * NS

D = 128
C = 128


def _dist_mult_body(si_hbm, pi_hbm, oi_hbm, nodes_hbm, rel_hbm, out_hbm,
                    si_v, pi_v, oi_v, s0, p0, o0, s1, p1, o1, out_v,
                    sem0, sem1):
    bpw = out_v.shape[0]
    nchunk = bpw // C
    wid = lax.axis_index("s") * NC + lax.axis_index("c")
    base = wid * bpw
    row_ids = lax.iota(jnp.int32, L)
    bufs = ((s0, p0, o0, sem0), (s1, p1, o1, sem1))

    pltpu.sync_copy(si_hbm.at[pl.ds(base, bpw)], si_v)
    pltpu.sync_copy(pi_hbm.at[pl.ds(base, bpw)], pi_v)
    pltpu.sync_copy(oi_hbm.at[pl.ds(base, bpw)], oi_v)

    def fire(c):
        s_b, p_b, o_b, sem = bufs[c % 2]
        return (
            pltpu.async_copy(nodes_hbm.at[si_v.at[pl.ds(c * C, C)]], s_b, sem),
            pltpu.async_copy(rel_hbm.at[pi_v.at[pl.ds(c * C, C)]], p_b, sem),
            pltpu.async_copy(nodes_hbm.at[oi_v.at[pl.ds(c * C, C)]], o_b, sem),
        )

    inflight = fire(0)
    for c in range(nchunk):
        for cp in inflight:
            cp.wait()
        if c + 1 < nchunk:
            inflight = fire(c + 1)
        s_b, p_b, o_b, _ = bufs[c % 2]

        def group_body(g, carry, c=c, s_b=s_b, p_b=p_b, o_b=o_b):
            rows = row_ids + g * L

            def dblock(db, accs):
                accs = list(accs)
                dbase = db * 32
                for u in range(32):
                    cols = (row_ids + u + dbase) & (D - 1)
                    sv = plsc.load_gather(s_b, [rows, cols])
                    pv = plsc.load_gather(p_b, [rows, cols])
                    ov = plsc.load_gather(o_b, [rows, cols])
                    accs[u % 4] = accs[u % 4] + sv * pv * ov
                return tuple(accs)

            zero = jnp.zeros((L,), jnp.float32)
            accs = lax.fori_loop(0, D // 32, dblock,
                                 (zero, zero, zero, zero))
            acc = (accs[0] + accs[1]) + (accs[2] + accs[3])
            out_v[pl.ds(c * C + g * L, L)] = acc
            return carry

        lax.fori_loop(0, C // L, group_body, 0)

    pltpu.sync_copy(out_v, out_hbm.at[pl.ds(base, bpw)])


def kernel(triples, nodes, relations):
    b = triples.shape[0]
    bpw = b // NW
    si = triples[:, 0].astype(jnp.int32)
    pi = triples[:, 1].astype(jnp.int32)
    oi = triples[:, 2].astype(jnp.int32)

    mesh = plsc.VectorSubcoreMesh(core_axis_name="c", subcore_axis_name="s")
    run = pl.kernel(
        _dist_mult_body,
        out_type=jax.ShapeDtypeStruct((b,), jnp.float32),
        mesh=mesh,
        compiler_params=pltpu.CompilerParams(needs_layout_passes=False),
        scratch_types=[
            pltpu.VMEM((bpw,), jnp.int32),
            pltpu.VMEM((bpw,), jnp.int32),
            pltpu.VMEM((bpw,), jnp.int32),
            pltpu.VMEM((C, D), jnp.float32),
            pltpu.VMEM((C, D), jnp.float32),
            pltpu.VMEM((C, D), jnp.float32),
            pltpu.VMEM((C, D), jnp.float32),
            pltpu.VMEM((C, D), jnp.float32),
            pltpu.VMEM((C, D), jnp.float32),
            pltpu.VMEM((bpw,), jnp.float32),
            pltpu.SemaphoreType.DMA,
            pltpu.SemaphoreType.DMA,
        ],
    )
    return run(si, pi, oi, nodes, relations)
